# baseline (speedup 1.0000x reference)
import functools

import jax
import jax.numpy as jnp
from jax import lax
from jax.experimental import pallas as pl
from jax.experimental.pallas import tpu as pltpu
from jax.experimental.pallas import tpu_sc as plsc

EMBED_DIM = 64
TOTAL = 16384 * 20
NUM_WORKERS = 32
BPW = TOTAL // NUM_WORKERS
CHUNK = 512
NCHUNK = BPW // CHUNK


def _emb_lookup(idx_flat, weight):
    mesh = plsc.VectorSubcoreMesh(core_axis_name="c", subcore_axis_name="s")

    @functools.partial(
        pl.kernel,
        mesh=mesh,
        out_type=jax.ShapeDtypeStruct((TOTAL, EMBED_DIM), jnp.float32),
        scratch_types=[
            pltpu.VMEM((BPW,), jnp.int32),
            pltpu.VMEM((CHUNK, EMBED_DIM), jnp.float32),
            pltpu.SemaphoreType.DMA,
        ],
        compiler_params=pltpu.CompilerParams(use_tc_tiling_on_sc=False),
    )
    def k(idx_hbm, table_hbm, out_hbm, idx_v, rows_v, gsem):
        wid = lax.axis_index("s") * 2 + lax.axis_index("c")
        base = wid * BPW
        pltpu.sync_copy(idx_hbm.at[pl.ds(base, BPW)], idx_v)

        def chunk_body(c, carry):
            start = c * CHUNK
            pltpu.async_copy(
                table_hbm.at[idx_v.at[pl.ds(start, CHUNK)]], rows_v, gsem
            ).wait()
            pltpu.sync_copy(rows_v, out_hbm.at[pl.ds(base + start, CHUNK)])
            return carry

        lax.fori_loop(0, NCHUNK, chunk_body, 0)

    return k(idx_flat, weight)


def kernel(input, weight):
    idx = input.reshape(-1).astype(jnp.int32)
    out = _emb_lookup(idx, weight)
    return out.reshape(input.shape + (EMBED_DIM,))


# speedup vs baseline: 1.2260x; 1.2260x over previous
import functools

import jax
import jax.numpy as jnp
from jax import lax
from jax.experimental import pallas as pl
from jax.experimental.pallas import tpu as pltpu
from jax.experimental.pallas import tpu_sc as plsc

EMBED_DIM = 64
TOTAL = 16384 * 20
NUM_WORKERS = 32
BPW = TOTAL // NUM_WORKERS
CHUNK = 512
NCHUNK = BPW // CHUNK


def _emb_lookup(idx_flat, weight):
    mesh = plsc.VectorSubcoreMesh(core_axis_name="c", subcore_axis_name="s")

    @functools.partial(
        pl.kernel,
        mesh=mesh,
        out_type=jax.ShapeDtypeStruct((TOTAL, EMBED_DIM), jnp.float32),
        scratch_types=[
            pltpu.VMEM((BPW,), jnp.int32),
            pltpu.VMEM((2, CHUNK, EMBED_DIM), jnp.float32),
            pltpu.SemaphoreType.DMA,
            pltpu.SemaphoreType.DMA,
        ],
        compiler_params=pltpu.CompilerParams(use_tc_tiling_on_sc=False),
    )
    def k(idx_hbm, table_hbm, out_hbm, idx_v, rows_v, gsem, wsem):
        wid = lax.axis_index("s") * 2 + lax.axis_index("c")
        base = wid * BPW
        pltpu.sync_copy(idx_hbm.at[pl.ds(base, BPW)], idx_v)

        def gather(c, b):
            return pltpu.async_copy(
                table_hbm.at[idx_v.at[pl.ds(c * CHUNK, CHUNK)]],
                rows_v.at[b], gsem)

        def write(c, b):
            return pltpu.async_copy(
                rows_v.at[b], out_hbm.at[pl.ds(base + c * CHUNK, CHUNK)], wsem)

        writes = [None] * NCHUNK
        g = gather(0, 0)
        for c in range(NCHUNK):
            b = c % 2
            if c + 1 < NCHUNK:
                if c >= 1:
                    writes[c - 1].wait()
                g_next = gather(c + 1, (c + 1) % 2)
            g.wait()
            writes[c] = write(c, b)
            if c + 1 < NCHUNK:
                g = g_next
        writes[NCHUNK - 2].wait()
        writes[NCHUNK - 1].wait()

    return k(idx_flat, weight)


def kernel(input, weight):
    idx = input.reshape(-1).astype(jnp.int32)
    out = _emb_lookup(idx, weight)
    return out.reshape(input.shape + (EMBED_DIM,))
